# Initial kernel scaffold; baseline (speedup 1.0000x reference)
import jax
import jax.numpy as jnp
from jax.experimental import pallas as pl


def kernel(lateral_inflows, lengths, slopes, manning_n, width_coef, width_exp, depth_coef, depth_exp, upstream_mask, is_outlet):
    raise NotImplementedError("write your pallas kernel here")


# speedup vs baseline: 289.8930x; 289.8930x over previous
import functools

import jax
import jax.numpy as jnp
from jax import lax
from jax.experimental import pallas as pl
from jax.experimental.pallas import tpu as pltpu
from jax.experimental.pallas import tpu_sc as plsc

N = 2048
T = 16
DT = 3600.0
L = 16
NCH = N // L
LN2 = 0.6931471805599453
SQRT2 = 1.4142135623730951


def _vlog(x):
    bits = lax.bitcast_convert_type(x, jnp.int32)
    e = (bits >> 23) - 127
    m = lax.bitcast_convert_type((bits & 0x007FFFFF) | 0x3F800000,
                                 jnp.float32)
    big = m >= SQRT2
    m = jnp.where(big, m * 0.5, m)
    e = jnp.where(big, e + 1, e)
    s = (m - 1.0) / (m + 1.0)
    z = s * s
    p = 1.0 + z * (1.0/3.0 + z * (1.0/5.0 + z * (1.0/7.0 + z * (1.0/9.0))))
    return e.astype(jnp.float32) * LN2 + 2.0 * s * p


def _routing_body(lat_hbm, len_hbm, slope_hbm, n_hbm, wc_hbm, we_hbm, dc_hbm,
                  de_hbm, ptr_hbm, alive_hbm, omask_hbm, out_hbm,
                  latv, lenv, sqrt_s, inv_n, ssl, wcv, wev, dcv, dev,
                  ptrv, alivev, omaskv,
                  Q, Qp, Ip, C0, v, u, wA, wB, pA, pB, outacc):
    cid = lax.axis_index("c")
    sid = lax.axis_index("s")

    @pl.when((cid == 0) & (sid == 0))
    def _work():
        pltpu.sync_copy(lat_hbm, latv)
        pltpu.sync_copy(len_hbm, lenv)
        pltpu.sync_copy(slope_hbm, sqrt_s)
        pltpu.sync_copy(n_hbm, inv_n)
        pltpu.sync_copy(wc_hbm, wcv)
        pltpu.sync_copy(we_hbm, wev)
        pltpu.sync_copy(dc_hbm, dcv)
        pltpu.sync_copy(de_hbm, dev)
        pltpu.sync_copy(ptr_hbm, ptrv)
        pltpu.sync_copy(alive_hbm, alivev)
        pltpu.sync_copy(omask_hbm, omaskv)

        def _pre(c, _):
            sl = pl.ds(c * L, L)
            slope_safe = jnp.maximum(sqrt_s[sl], 1e-6)
            sqrt_s[sl] = jnp.exp(0.5 * _vlog(slope_safe))
            inv_n[sl] = 1.0 / jnp.maximum(inv_n[sl], 0.001)
            ssl[sl] = slope_safe * lenv[sl]
            Q[sl] = jnp.full((L,), 0.1, jnp.float32)
            Qp[sl] = jnp.full((L,), 0.1, jnp.float32)
            Ip[sl] = jnp.zeros((L,), jnp.float32)
            return 0
        lax.fori_loop(0, NCH, _pre, 0, unroll=False)
        outacc[:] = jnp.zeros((L,), jnp.float32)

        def _timestep(t, _):
            def _params(c, _):
                sl = pl.ds(c * L, L)
                Qr = jnp.maximum(Q[sl], 0.1)
                logQ = _vlog(Qr)
                width = wcv[sl] * jnp.exp(wev[sl] * logQ)
                depth = dcv[sl] * jnp.exp(dev[sl] * logQ)
                Rh = (width * depth) / (width + 2.0 * depth)
                V = inv_n[sl] * jnp.exp((2.0 / 3.0) * _vlog(Rh)) * sqrt_s[sl]
                cel = jnp.maximum((5.0 / 3.0) * V, 0.01)
                K = jnp.maximum(lenv[sl] / cel, DT * 0.1)
                X = 0.5 * (1.0 - Qr / (width * cel * ssl[sl] + 1e-6))
                X = jnp.clip(X, 0.0, 0.5)
                kx2 = 2.0 * K * X
                d2 = 2.0 * K * (1.0 - X)
                denom = d2 + DT
                c0 = jnp.maximum((DT - kx2) / denom, 0.0)
                c1 = jnp.maximum((DT + kx2) / denom, 0.0)
                c2 = jnp.maximum((d2 - DT) / denom, 0.0)
                tot = c0 + c1 + c2
                c0 = c0 / tot
                C0[sl] = c0
                v[sl] = (c0 * latv[t, sl] + (c1 / tot) * Ip[sl]
                         + (c2 / tot) * Qp[sl])
                return 0
            lax.fori_loop(0, NCH, _params, 0, unroll=False)

            def _winit(c, _):
                sl = pl.ds(c * L, L)
                pc = ptrv[sl]
                wA[sl] = plsc.load_gather(C0, [pc]) * alivev[sl]
                pA[sl] = pc
                return 0
            lax.fori_loop(0, NCH, _winit, 0, unroll=False)

            for k in range(11):
                wsrc, psrc = (wA, pA) if k % 2 == 0 else (wB, pB)
                wdst, pdst = (wB, pB) if k % 2 == 0 else (wA, pA)

                def _zero(c, _):
                    u[pl.ds(c * L, L)] = jnp.zeros((L,), jnp.float32)
                    return 0
                lax.fori_loop(0, NCH, _zero, 0, unroll=False)

                def _scatter(c, _, wsrc=wsrc, psrc=psrc):
                    sl = pl.ds(c * L, L)
                    plsc.addupdate_scatter(u, [psrc[sl]], wsrc[sl] * v[sl])
                    return 0
                lax.fori_loop(0, NCH, _scatter, 0, unroll=False)

                if k < 10:
                    def _combine(c, _, wsrc=wsrc, psrc=psrc, wdst=wdst,
                                 pdst=pdst):
                        sl = pl.ds(c * L, L)
                        v[sl] = v[sl] + u[sl]
                        pc = psrc[sl]
                        wdst[sl] = wsrc[sl] * plsc.load_gather(wsrc, [pc])
                        pdst[sl] = plsc.load_gather(psrc, [pc])
                        return 0
                    lax.fori_loop(0, NCH, _combine, 0, unroll=False)
                else:
                    def _final(c, _):
                        sl = pl.ds(c * L, L)
                        v[sl] = v[sl] + u[sl]
                        return 0
                    lax.fori_loop(0, NCH, _final, 0, unroll=False)

            def _zero2(c, _):
                u[pl.ds(c * L, L)] = jnp.zeros((L,), jnp.float32)
                return 0
            lax.fori_loop(0, NCH, _zero2, 0, unroll=False)

            def _ups(c, _):
                sl = pl.ds(c * L, L)
                plsc.addupdate_scatter(u, [ptrv[sl]], alivev[sl] * v[sl])
                return 0
            lax.fori_loop(0, NCH, _ups, 0, unroll=False)

            def _update(c, acc):
                sl = pl.ds(c * L, L)
                vc = v[sl]
                acc = acc + omaskv[sl] * vc
                Ip[sl] = u[sl] + latv[t, sl]
                Qp[sl] = Q[sl]
                Q[sl] = vc
                return acc
            acc = lax.fori_loop(0, NCH, _update,
                                jnp.zeros((L,), jnp.float32), unroll=False)
            s = jnp.sum(acc, axis=0)
            tlane = lax.iota(jnp.int32, L) == t
            outacc[:] = outacc[:] + jnp.where(tlane, s, 0.0)
            return 0

        lax.fori_loop(0, T, _timestep, 0, unroll=False)
        pltpu.sync_copy(outacc, out_hbm)


def kernel(lateral_inflows, lengths, slopes, manning_n, width_coef, width_exp,
           depth_coef, depth_exp, upstream_mask, is_outlet):
    ptr = jnp.argmax(upstream_mask, axis=0).astype(jnp.int32)
    alive = jnp.any(upstream_mask, axis=0).astype(jnp.float32)
    omask = is_outlet.astype(jnp.float32)

    f32v = pltpu.VMEM((N,), jnp.float32)
    i32v = pltpu.VMEM((N,), jnp.int32)
    run = pl.kernel(
        _routing_body,
        out_type=jax.ShapeDtypeStruct((T,), jnp.float32),
        mesh=plsc.VectorSubcoreMesh(core_axis_name="c", subcore_axis_name="s"),
        compiler_params=pltpu.CompilerParams(needs_layout_passes=False),
        scratch_types=[
            pltpu.VMEM((T, N), jnp.float32),
            f32v, f32v, f32v, f32v,
            f32v, f32v, f32v, f32v,
            i32v, f32v, f32v,
            f32v, f32v, f32v, f32v,
            f32v, f32v,
            f32v, f32v, i32v, i32v,
            pltpu.VMEM((L,), jnp.float32),
        ],
    )
    return run(lateral_inflows.astype(jnp.float32), lengths, slopes, manning_n,
               width_coef, width_exp, depth_coef, depth_exp, ptr, alive, omask)


# speedup vs baseline: 290.6127x; 1.0025x over previous
import functools

import jax
import jax.numpy as jnp
from jax import lax
from jax.experimental import pallas as pl
from jax.experimental.pallas import tpu as pltpu
from jax.experimental.pallas import tpu_sc as plsc

N = 2048
T = 16
DT = 3600.0
L = 16
NCH = N // L
LN2 = 0.6931471805599453
SQRT2 = 1.4142135623730951


def _vlog(x):
    bits = lax.bitcast_convert_type(x, jnp.int32)
    e = (bits >> 23) - 127
    m = lax.bitcast_convert_type((bits & 0x007FFFFF) | 0x3F800000,
                                 jnp.float32)
    big = m >= SQRT2
    m = jnp.where(big, m * 0.5, m)
    e = jnp.where(big, e + 1, e)
    s = (m - 1.0) / (m + 1.0)
    z = s * s
    p = 1.0 + z * (1.0/3.0 + z * (1.0/5.0 + z * (1.0/7.0 + z * (1.0/9.0))))
    return e.astype(jnp.float32) * LN2 + 2.0 * s * p


def _routing_body(lat_hbm, len_hbm, slope_hbm, n_hbm, wc_hbm, we_hbm, dc_hbm,
                  de_hbm, ptr_hbm, alive_hbm, omask_hbm, out_hbm,
                  latv, lenv, sqrt_s, inv_n, ssl, wcv, wev, dcv, dev,
                  ptrv, alivev, omaskv,
                  Q, Qp, Ip, C0, v, v2, u, wA, wB, pA, pB, outacc, flagv):
    cid = lax.axis_index("c")
    sid = lax.axis_index("s")

    @pl.when((cid == 0) & (sid == 0))
    def _work():
        pltpu.sync_copy(lat_hbm, latv)
        pltpu.sync_copy(len_hbm, lenv)
        pltpu.sync_copy(slope_hbm, sqrt_s)
        pltpu.sync_copy(n_hbm, inv_n)
        pltpu.sync_copy(wc_hbm, wcv)
        pltpu.sync_copy(we_hbm, wev)
        pltpu.sync_copy(dc_hbm, dcv)
        pltpu.sync_copy(de_hbm, dev)
        pltpu.sync_copy(ptr_hbm, ptrv)
        pltpu.sync_copy(alive_hbm, alivev)
        pltpu.sync_copy(omask_hbm, omaskv)

        def _pre(c, _):
            sl = pl.ds(c * L, L)
            slope_safe = jnp.maximum(sqrt_s[sl], 1e-6)
            sqrt_s[sl] = jnp.exp(0.5 * _vlog(slope_safe))
            inv_n[sl] = 1.0 / jnp.maximum(inv_n[sl], 0.001)
            ssl[sl] = slope_safe * lenv[sl]
            Q[sl] = jnp.full((L,), 0.1, jnp.float32)
            Qp[sl] = jnp.full((L,), 0.1, jnp.float32)
            Ip[sl] = jnp.zeros((L,), jnp.float32)
            return 0
        lax.fori_loop(0, NCH, _pre, 0, unroll=2)
        outacc[:] = jnp.zeros((L,), jnp.float32)

        def _timestep(t, _):
            def _params(c, _):
                sl = pl.ds(c * L, L)
                Qr = jnp.maximum(Q[sl], 0.1)
                logQ = _vlog(Qr)
                width = wcv[sl] * jnp.exp(wev[sl] * logQ)
                depth = dcv[sl] * jnp.exp(dev[sl] * logQ)
                Rh = (width * depth) / (width + 2.0 * depth)
                V = inv_n[sl] * jnp.exp((2.0 / 3.0) * _vlog(Rh)) * sqrt_s[sl]
                cel = jnp.maximum((5.0 / 3.0) * V, 0.01)
                K = jnp.maximum(lenv[sl] / cel, DT * 0.1)
                X = 0.5 * (1.0 - Qr / (width * cel * ssl[sl] + 1e-6))
                X = jnp.clip(X, 0.0, 0.5)
                kx2 = 2.0 * K * X
                d2 = 2.0 * K * (1.0 - X)
                denom = d2 + DT
                c0 = jnp.maximum((DT - kx2) / denom, 0.0)
                c1 = jnp.maximum((DT + kx2) / denom, 0.0)
                c2 = jnp.maximum((d2 - DT) / denom, 0.0)
                tot = c0 + c1 + c2
                c0 = c0 / tot
                C0[sl] = c0
                b = (c0 * latv[t, sl] + (c1 / tot) * Ip[sl]
                     + (c2 / tot) * Qp[sl])
                v[sl] = b
                v2[sl] = b
                return 0
            lax.fori_loop(0, NCH, _params, 0, unroll=2)

            def _winit(c, _):
                sl = pl.ds(c * L, L)
                pc = ptrv[sl]
                wA[sl] = plsc.load_gather(C0, [pc]) * alivev[sl]
                pA[sl] = pc
                return 0
            lax.fori_loop(0, NCH, _winit, 0, unroll=4)

            def _copy(c, _, vsrc, vdst):
                sl = pl.ds(c * L, L)
                vdst[sl] = vsrc[sl]
                return 0

            def _fused(c, acc, vsrc, vdst, wsrc, psrc, wdst, pdst):
                sl = pl.ds(c * L, L)
                pc = psrc[sl]
                wc = wsrc[sl]
                plsc.addupdate_scatter(vdst, [pc], wc * vsrc[sl])
                w2 = wc * plsc.load_gather(wsrc, [pc])
                wdst[sl] = w2
                pdst[sl] = plsc.load_gather(psrc, [pc])
                return acc + w2
            for k in range(6):
                vsrc, vdst = (v, v2) if k % 2 == 0 else (v2, v)
                wsrc, psrc = (wA, pA) if k % 2 == 0 else (wB, pB)
                wdst, pdst = (wB, pB) if k % 2 == 0 else (wA, pA)
                if k > 0:
                    lax.fori_loop(
                        0, NCH,
                        functools.partial(_copy, vsrc=vsrc, vdst=vdst),
                        0, unroll=4)
                acc = lax.fori_loop(
                    0, NCH,
                    functools.partial(_fused, vsrc=vsrc, vdst=vdst,
                                      wsrc=wsrc, psrc=psrc,
                                      wdst=wdst, pdst=pdst),
                    jnp.zeros((L,), jnp.float32), unroll=4)
                if k == 5:
                    flagv[:] = acc

            for k in range(6, 11):
                wsrc, psrc = (wA, pA) if k % 2 == 0 else (wB, pB)
                wdst, pdst = (wB, pB) if k % 2 == 0 else (wA, pA)
                g = jnp.sum(flagv[:], axis=0) > 0.0

                @pl.when(g)
                def _round(k=k, wsrc=wsrc, psrc=psrc, wdst=wdst, pdst=pdst):
                    def _zero(c, _):
                        u[pl.ds(c * L, L)] = jnp.zeros((L,), jnp.float32)
                        return 0
                    lax.fori_loop(0, NCH, _zero, 0, unroll=4)

                    if k < 10:
                        def _sc(c, acc):
                            sl = pl.ds(c * L, L)
                            pc = psrc[sl]
                            wc = wsrc[sl]
                            plsc.addupdate_scatter(u, [pc], wc * v[sl])
                            w2 = wc * plsc.load_gather(wsrc, [pc])
                            wdst[sl] = w2
                            pdst[sl] = plsc.load_gather(psrc, [pc])
                            return acc + w2
                        acc = lax.fori_loop(0, NCH, _sc,
                                            jnp.zeros((L,), jnp.float32),
                                            unroll=4)
                        flagv[:] = acc
                    else:
                        def _sc_last(c, _):
                            sl = pl.ds(c * L, L)
                            plsc.addupdate_scatter(
                                u, [psrc[sl]], wsrc[sl] * v[sl])
                            return 0
                        lax.fori_loop(0, NCH, _sc_last, 0, unroll=4)

                    def _add(c, _):
                        sl = pl.ds(c * L, L)
                        v[sl] = v[sl] + u[sl]
                        return 0
                    lax.fori_loop(0, NCH, _add, 0, unroll=4)

            def _lat(c, _):
                sl = pl.ds(c * L, L)
                u[sl] = latv[t, sl]
                return 0
            lax.fori_loop(0, NCH, _lat, 0, unroll=4)

            def _ups(c, _):
                sl = pl.ds(c * L, L)
                plsc.addupdate_scatter(u, [ptrv[sl]], alivev[sl] * v[sl])
                return 0
            lax.fori_loop(0, NCH, _ups, 0, unroll=4)

            def _update(c, acc):
                sl = pl.ds(c * L, L)
                vc = v[sl]
                acc = acc + omaskv[sl] * vc
                Ip[sl] = u[sl]
                Qp[sl] = Q[sl]
                Q[sl] = vc
                return acc
            acc = lax.fori_loop(0, NCH, _update,
                                jnp.zeros((L,), jnp.float32), unroll=4)
            s = jnp.sum(acc, axis=0)
            tlane = lax.iota(jnp.int32, L) == t
            outacc[:] = outacc[:] + jnp.where(tlane, s, 0.0)
            return 0

        lax.fori_loop(0, T, _timestep, 0, unroll=False)
        pltpu.sync_copy(outacc, out_hbm)


def kernel(lateral_inflows, lengths, slopes, manning_n, width_coef, width_exp,
           depth_coef, depth_exp, upstream_mask, is_outlet):
    ptr = jnp.argmax(upstream_mask, axis=0).astype(jnp.int32)
    alive = jnp.any(upstream_mask, axis=0).astype(jnp.float32)
    omask = is_outlet.astype(jnp.float32)

    f32v = pltpu.VMEM((N,), jnp.float32)
    i32v = pltpu.VMEM((N,), jnp.int32)
    run = pl.kernel(
        _routing_body,
        out_type=jax.ShapeDtypeStruct((T,), jnp.float32),
        mesh=plsc.VectorSubcoreMesh(core_axis_name="c", subcore_axis_name="s"),
        compiler_params=pltpu.CompilerParams(needs_layout_passes=False),
        scratch_types=[
            pltpu.VMEM((T, N), jnp.float32),
            f32v, f32v, f32v, f32v,
            f32v, f32v, f32v, f32v,
            i32v, f32v, f32v,
            f32v, f32v, f32v, f32v,
            f32v, f32v, f32v,
            f32v, f32v, i32v, i32v,
            pltpu.VMEM((L,), jnp.float32),
            pltpu.VMEM((L,), jnp.float32),
        ],
    )
    return run(lateral_inflows.astype(jnp.float32), lengths, slopes, manning_n,
               width_coef, width_exp, depth_coef, depth_exp, ptr, alive, omask)


# speedup vs baseline: 879.8279x; 3.0275x over previous
import functools

import jax
import jax.numpy as jnp
from jax import lax
from jax.experimental import pallas as pl
from jax.experimental.pallas import tpu as pltpu
from jax.experimental.pallas import tpu_sc as plsc

N = 2048
T = 16
DT = 3600.0
L = 16
LN2 = 0.6931471805599453
SQRT2 = 1.4142135623730951


def _vlog(x):
    bits = lax.bitcast_convert_type(x, jnp.int32)
    e = (bits >> 23) - 127
    m = lax.bitcast_convert_type((bits & 0x007FFFFF) | 0x3F800000,
                                 jnp.float32)
    big = m >= SQRT2
    m = jnp.where(big, m * 0.5, m)
    e = jnp.where(big, e + 1, e)
    s = (m - 1.0) / (m + 1.0)
    z = s * s
    p = 1.0 + z * (1.0/3.0 + z * (1.0/5.0 + z * (1.0/7.0 + z * (1.0/9.0))))
    return e.astype(jnp.float32) * LN2 + 2.0 * s * p


def _sweep(body, *, carry=None, unroll=4):
    if carry is None:
        def wrapped(i, j):
            body(i)
            return j
        plsc.parallel_loop(0, N, step=L, unroll=unroll,
                           carry=jnp.int32(0))(wrapped)
        return None
    return plsc.parallel_loop(0, N, step=L, unroll=unroll, carry=carry)(body)


def _routing_body(lat_hbm, len_hbm, slope_hbm, n_hbm, wc_hbm, we_hbm, dc_hbm,
                  de_hbm, ptr_hbm, alive_hbm, omask_hbm, out_hbm,
                  latv, lenv, sqrt_s, inv_n, ssl, wcv, wev, dcv, dev,
                  ptrv, alivev, omaskv,
                  Q, Qp, Ip, C0, v, v2, u, wA, wB, pA, pB, outacc, flagv):
    cid = lax.axis_index("c")
    sid = lax.axis_index("s")

    @pl.when((cid == 0) & (sid == 0))
    def _work():
        pltpu.sync_copy(lat_hbm, latv)
        pltpu.sync_copy(len_hbm, lenv)
        pltpu.sync_copy(slope_hbm, sqrt_s)
        pltpu.sync_copy(n_hbm, inv_n)
        pltpu.sync_copy(wc_hbm, wcv)
        pltpu.sync_copy(we_hbm, wev)
        pltpu.sync_copy(dc_hbm, dcv)
        pltpu.sync_copy(de_hbm, dev)
        pltpu.sync_copy(ptr_hbm, ptrv)
        pltpu.sync_copy(alive_hbm, alivev)
        pltpu.sync_copy(omask_hbm, omaskv)

        def _pre(i):
            sl = pl.ds(i, L)
            slope_safe = jnp.maximum(sqrt_s[sl], 1e-6)
            sqrt_s[sl] = jnp.exp(0.5 * _vlog(slope_safe))
            inv_n[sl] = 1.0 / jnp.maximum(inv_n[sl], 0.001)
            ssl[sl] = slope_safe * lenv[sl]
            Q[sl] = jnp.full((L,), 0.1, jnp.float32)
            Qp[sl] = jnp.full((L,), 0.1, jnp.float32)
            Ip[sl] = jnp.zeros((L,), jnp.float32)
        _sweep(_pre, unroll=2)
        outacc[:] = jnp.zeros((L,), jnp.float32)

        def _timestep(t, _):
            def _params(i):
                sl = pl.ds(i, L)
                Qr = jnp.maximum(Q[sl], 0.1)
                logQ = _vlog(Qr)
                width = wcv[sl] * jnp.exp(wev[sl] * logQ)
                depth = dcv[sl] * jnp.exp(dev[sl] * logQ)
                Rh = (width * depth) / (width + 2.0 * depth)
                V = inv_n[sl] * jnp.exp((2.0 / 3.0) * _vlog(Rh)) * sqrt_s[sl]
                cel = jnp.maximum((5.0 / 3.0) * V, 0.01)
                K = jnp.maximum(lenv[sl] / cel, DT * 0.1)
                X = 0.5 * (1.0 - Qr / (width * cel * ssl[sl] + 1e-6))
                X = jnp.clip(X, 0.0, 0.5)
                kx2 = 2.0 * K * X
                d2 = 2.0 * K * (1.0 - X)
                denom = d2 + DT
                c0 = jnp.maximum((DT - kx2) / denom, 0.0)
                c1 = jnp.maximum((DT + kx2) / denom, 0.0)
                c2 = jnp.maximum((d2 - DT) / denom, 0.0)
                tot = c0 + c1 + c2
                c0 = c0 / tot
                C0[sl] = c0
                b = (c0 * latv[t, sl] + (c1 / tot) * Ip[sl]
                     + (c2 / tot) * Qp[sl])
                v[sl] = b
                v2[sl] = b
            _sweep(_params, unroll=2)

            def _winit(i):
                sl = pl.ds(i, L)
                pc = ptrv[sl]
                wA[sl] = plsc.load_gather(C0, [pc]) * alivev[sl]
                pA[sl] = pc
            _sweep(_winit)

            for k in range(6):
                vsrc, vdst = (v, v2) if k % 2 == 0 else (v2, v)
                wsrc, psrc = (wA, pA) if k % 2 == 0 else (wB, pB)
                wdst, pdst = (wB, pB) if k % 2 == 0 else (wA, pA)
                if k > 0:
                    def _copy(i, vsrc=vsrc, vdst=vdst):
                        sl = pl.ds(i, L)
                        vdst[sl] = vsrc[sl]
                    _sweep(_copy)

                def _fused(i, acc, vsrc=vsrc, vdst=vdst, wsrc=wsrc,
                           psrc=psrc, wdst=wdst, pdst=pdst):
                    sl = pl.ds(i, L)
                    pc = psrc[sl]
                    wc = wsrc[sl]
                    plsc.addupdate_scatter(vdst, [pc], wc * vsrc[sl])
                    w2 = wc * plsc.load_gather(wsrc, [pc])
                    wdst[sl] = w2
                    pdst[sl] = plsc.load_gather(psrc, [pc])
                    return acc + w2
                acc = _sweep(_fused, carry=jnp.zeros((L,), jnp.float32))
                if k == 5:
                    flagv[:] = acc

            for k in range(6, 11):
                wsrc, psrc = (wA, pA) if k % 2 == 0 else (wB, pB)
                wdst, pdst = (wB, pB) if k % 2 == 0 else (wA, pA)
                g = jnp.sum(flagv[:], axis=0) > 0.0

                @pl.when(g)
                def _round(k=k, wsrc=wsrc, psrc=psrc, wdst=wdst, pdst=pdst):
                    def _zero(i):
                        u[pl.ds(i, L)] = jnp.zeros((L,), jnp.float32)
                    _sweep(_zero)

                    if k < 10:
                        def _sc(i, acc):
                            sl = pl.ds(i, L)
                            pc = psrc[sl]
                            wc = wsrc[sl]
                            plsc.addupdate_scatter(u, [pc], wc * v[sl])
                            w2 = wc * plsc.load_gather(wsrc, [pc])
                            wdst[sl] = w2
                            pdst[sl] = plsc.load_gather(psrc, [pc])
                            return acc + w2
                        acc = _sweep(_sc, carry=jnp.zeros((L,), jnp.float32))
                        flagv[:] = acc
                    else:
                        def _sc_last(i):
                            sl = pl.ds(i, L)
                            plsc.addupdate_scatter(
                                u, [psrc[sl]], wsrc[sl] * v[sl])
                        _sweep(_sc_last)

                    def _add(i):
                        sl = pl.ds(i, L)
                        v[sl] = v[sl] + u[sl]
                    _sweep(_add)

            def _lat(i):
                sl = pl.ds(i, L)
                u[sl] = latv[t, sl]
            _sweep(_lat)

            def _ups(i):
                sl = pl.ds(i, L)
                plsc.addupdate_scatter(u, [ptrv[sl]], alivev[sl] * v[sl])
            _sweep(_ups)

            def _update(i, acc):
                sl = pl.ds(i, L)
                vc = v[sl]
                acc = acc + omaskv[sl] * vc
                Ip[sl] = u[sl]
                Qp[sl] = Q[sl]
                Q[sl] = vc
                return acc
            acc = _sweep(_update, carry=jnp.zeros((L,), jnp.float32))
            s = jnp.sum(acc, axis=0)
            tlane = lax.iota(jnp.int32, L) == t
            outacc[:] = outacc[:] + jnp.where(tlane, s, 0.0)
            return 0

        lax.fori_loop(0, T, _timestep, 0, unroll=False)
        pltpu.sync_copy(outacc, out_hbm)


def kernel(lateral_inflows, lengths, slopes, manning_n, width_coef, width_exp,
           depth_coef, depth_exp, upstream_mask, is_outlet):
    ptr = jnp.argmax(upstream_mask, axis=0).astype(jnp.int32)
    alive = jnp.any(upstream_mask, axis=0).astype(jnp.float32)
    omask = is_outlet.astype(jnp.float32)

    f32v = pltpu.VMEM((N,), jnp.float32)
    i32v = pltpu.VMEM((N,), jnp.int32)
    run = pl.kernel(
        _routing_body,
        out_type=jax.ShapeDtypeStruct((T,), jnp.float32),
        mesh=plsc.VectorSubcoreMesh(core_axis_name="c", subcore_axis_name="s"),
        compiler_params=pltpu.CompilerParams(needs_layout_passes=False),
        scratch_types=[
            pltpu.VMEM((T, N), jnp.float32),
            f32v, f32v, f32v, f32v,
            f32v, f32v, f32v, f32v,
            i32v, f32v, f32v,
            f32v, f32v, f32v, f32v,
            f32v, f32v, f32v,
            f32v, f32v, i32v, i32v,
            pltpu.VMEM((L,), jnp.float32),
            pltpu.VMEM((L,), jnp.float32),
        ],
    )
    return run(lateral_inflows.astype(jnp.float32), lengths, slopes, manning_n,
               width_coef, width_exp, depth_coef, depth_exp, ptr, alive, omask)


# speedup vs baseline: 887.3312x; 1.0085x over previous
import functools

import jax
import jax.numpy as jnp
from jax import lax
from jax.experimental import pallas as pl
from jax.experimental.pallas import tpu as pltpu
from jax.experimental.pallas import tpu_sc as plsc

N = 2048
T = 16
DT = 3600.0
L = 16
LN2 = 0.6931471805599453
SQRT2 = 1.4142135623730951


def _vlog(x):
    bits = lax.bitcast_convert_type(x, jnp.int32)
    e = (bits >> 23) - 127
    m = lax.bitcast_convert_type((bits & 0x007FFFFF) | 0x3F800000,
                                 jnp.float32)
    big = m >= SQRT2
    m = jnp.where(big, m * 0.5, m)
    e = jnp.where(big, e + 1, e)
    s = (m - 1.0) * (1.0 / (m + 1.0))
    z = s * s
    p = 1.0 + z * (1.0/3.0 + z * (1.0/5.0 + z * (1.0/7.0 + z * (1.0/9.0))))
    return e.astype(jnp.float32) * LN2 + 2.0 * s * p


def _sweep(body, *, carry=None, unroll=8):
    if carry is None:
        def wrapped(i, j):
            body(i)
            return j
        plsc.parallel_loop(0, N, step=L, unroll=unroll,
                           carry=jnp.int32(0))(wrapped)
        return None
    return plsc.parallel_loop(0, N, step=L, unroll=unroll, carry=carry)(body)


def _routing_body(lat_hbm, len_hbm, slope_hbm, n_hbm, wc_hbm, we_hbm, dc_hbm,
                  de_hbm, ptr_hbm, alive_hbm, omask_hbm, out_hbm,
                  latv, lenv, sqrt_s, inv_n, ssl, wcv, wev, dcv, dev,
                  ptrv, alivev, omaskv,
                  Q, Qp, Ip, C0, v, v2, u, wA, wB, pA, pB, outacc, flagv):
    cid = lax.axis_index("c")
    sid = lax.axis_index("s")

    @pl.when((cid == 0) & (sid == 0))
    def _work():
        pltpu.sync_copy(lat_hbm, latv)
        pltpu.sync_copy(len_hbm, lenv)
        pltpu.sync_copy(slope_hbm, sqrt_s)
        pltpu.sync_copy(n_hbm, inv_n)
        pltpu.sync_copy(wc_hbm, wcv)
        pltpu.sync_copy(we_hbm, wev)
        pltpu.sync_copy(dc_hbm, dcv)
        pltpu.sync_copy(de_hbm, dev)
        pltpu.sync_copy(ptr_hbm, ptrv)
        pltpu.sync_copy(alive_hbm, alivev)
        pltpu.sync_copy(omask_hbm, omaskv)

        def _pre(i):
            sl = pl.ds(i, L)
            slope_safe = jnp.maximum(sqrt_s[sl], 1e-6)
            sqrt_s[sl] = jnp.exp(0.5 * _vlog(slope_safe))
            inv_n[sl] = 1.0 / jnp.maximum(inv_n[sl], 0.001)
            ssl[sl] = slope_safe * lenv[sl]
            Q[sl] = jnp.full((L,), 0.1, jnp.float32)
            Qp[sl] = jnp.full((L,), 0.1, jnp.float32)
            Ip[sl] = jnp.zeros((L,), jnp.float32)
        _sweep(_pre, unroll=2)
        outacc[:] = jnp.zeros((L,), jnp.float32)

        def _timestep(t, _):
            def _params(i):
                sl = pl.ds(i, L)
                Qr = jnp.maximum(Q[sl], 0.1)
                logQ = _vlog(Qr)
                width = wcv[sl] * jnp.exp(wev[sl] * logQ)
                depth = dcv[sl] * jnp.exp(dev[sl] * logQ)
                Rh = (width * depth) * (1.0 / (width + 2.0 * depth))
                V = inv_n[sl] * jnp.exp((2.0 / 3.0) * _vlog(Rh)) * sqrt_s[sl]
                cel = jnp.maximum((5.0 / 3.0) * V, 0.01)
                K = jnp.maximum(lenv[sl] * (1.0 / cel), DT * 0.1)
                X = 0.5 * (1.0 - Qr * (1.0 / (width * cel * ssl[sl] + 1e-6)))
                X = jnp.clip(X, 0.0, 0.5)
                kx2 = 2.0 * K * X
                d2 = 2.0 * K * (1.0 - X)
                rden = 1.0 / (d2 + DT)
                c0 = jnp.maximum((DT - kx2) * rden, 0.0)
                c1 = jnp.maximum((DT + kx2) * rden, 0.0)
                c2 = jnp.maximum((d2 - DT) * rden, 0.0)
                rtot = 1.0 / (c0 + c1 + c2)
                c0 = c0 * rtot
                C0[sl] = c0
                b = (c0 * latv[t, sl] + (c1 * rtot) * Ip[sl]
                     + (c2 * rtot) * Qp[sl])
                v[sl] = b
                v2[sl] = b
            _sweep(_params, unroll=4)

            def _winit(i):
                sl = pl.ds(i, L)
                pc = ptrv[sl]
                wA[sl] = plsc.load_gather(C0, [pc]) * alivev[sl]
                pA[sl] = pc
            _sweep(_winit)

            for k in range(6):
                vsrc, vdst = (v, v2) if k % 2 == 0 else (v2, v)
                wsrc, psrc = (wA, pA) if k % 2 == 0 else (wB, pB)
                wdst, pdst = (wB, pB) if k % 2 == 0 else (wA, pA)
                if k > 0:
                    def _copy(i, vsrc=vsrc, vdst=vdst):
                        sl = pl.ds(i, L)
                        vdst[sl] = vsrc[sl]
                    _sweep(_copy)

                def _fused(i, acc, vsrc=vsrc, vdst=vdst, wsrc=wsrc,
                           psrc=psrc, wdst=wdst, pdst=pdst):
                    sl = pl.ds(i, L)
                    pc = psrc[sl]
                    wc = wsrc[sl]
                    plsc.addupdate_scatter(vdst, [pc], wc * vsrc[sl])
                    w2 = wc * plsc.load_gather(wsrc, [pc])
                    wdst[sl] = w2
                    pdst[sl] = plsc.load_gather(psrc, [pc])
                    return acc + w2
                acc = _sweep(_fused, carry=jnp.zeros((L,), jnp.float32))
                if k == 5:
                    flagv[:] = acc

            for k in range(6, 11):
                wsrc, psrc = (wA, pA) if k % 2 == 0 else (wB, pB)
                wdst, pdst = (wB, pB) if k % 2 == 0 else (wA, pA)
                g = jnp.sum(flagv[:], axis=0) > 0.0

                @pl.when(g)
                def _round(k=k, wsrc=wsrc, psrc=psrc, wdst=wdst, pdst=pdst):
                    def _zero(i):
                        u[pl.ds(i, L)] = jnp.zeros((L,), jnp.float32)
                    _sweep(_zero)

                    if k < 10:
                        def _sc(i, acc):
                            sl = pl.ds(i, L)
                            pc = psrc[sl]
                            wc = wsrc[sl]
                            plsc.addupdate_scatter(u, [pc], wc * v[sl])
                            w2 = wc * plsc.load_gather(wsrc, [pc])
                            wdst[sl] = w2
                            pdst[sl] = plsc.load_gather(psrc, [pc])
                            return acc + w2
                        acc = _sweep(_sc, carry=jnp.zeros((L,), jnp.float32))
                        flagv[:] = acc
                    else:
                        def _sc_last(i):
                            sl = pl.ds(i, L)
                            plsc.addupdate_scatter(
                                u, [psrc[sl]], wsrc[sl] * v[sl])
                        _sweep(_sc_last)

                    def _add(i):
                        sl = pl.ds(i, L)
                        v[sl] = v[sl] + u[sl]
                    _sweep(_add)

            def _lat(i):
                sl = pl.ds(i, L)
                u[sl] = latv[t, sl]
            _sweep(_lat)

            def _ups(i):
                sl = pl.ds(i, L)
                plsc.addupdate_scatter(u, [ptrv[sl]], alivev[sl] * v[sl])
            _sweep(_ups)

            def _update(i, acc):
                sl = pl.ds(i, L)
                vc = v[sl]
                acc = acc + omaskv[sl] * vc
                Ip[sl] = u[sl]
                Qp[sl] = Q[sl]
                Q[sl] = vc
                return acc
            acc = _sweep(_update, carry=jnp.zeros((L,), jnp.float32))
            s = jnp.sum(acc, axis=0)
            tlane = lax.iota(jnp.int32, L) == t
            outacc[:] = outacc[:] + jnp.where(tlane, s, 0.0)
            return 0

        lax.fori_loop(0, T, _timestep, 0, unroll=False)
        pltpu.sync_copy(outacc, out_hbm)


def kernel(lateral_inflows, lengths, slopes, manning_n, width_coef, width_exp,
           depth_coef, depth_exp, upstream_mask, is_outlet):
    ptr = jnp.argmax(upstream_mask, axis=0).astype(jnp.int32)
    alive = jnp.any(upstream_mask, axis=0).astype(jnp.float32)
    omask = is_outlet.astype(jnp.float32)

    f32v = pltpu.VMEM((N,), jnp.float32)
    i32v = pltpu.VMEM((N,), jnp.int32)
    run = pl.kernel(
        _routing_body,
        out_type=jax.ShapeDtypeStruct((T,), jnp.float32),
        mesh=plsc.VectorSubcoreMesh(core_axis_name="c", subcore_axis_name="s"),
        compiler_params=pltpu.CompilerParams(needs_layout_passes=False),
        scratch_types=[
            pltpu.VMEM((T, N), jnp.float32),
            f32v, f32v, f32v, f32v,
            f32v, f32v, f32v, f32v,
            i32v, f32v, f32v,
            f32v, f32v, f32v, f32v,
            f32v, f32v, f32v,
            f32v, f32v, i32v, i32v,
            pltpu.VMEM((L,), jnp.float32),
            pltpu.VMEM((L,), jnp.float32),
        ],
    )
    return run(lateral_inflows.astype(jnp.float32), lengths, slopes, manning_n,
               width_coef, width_exp, depth_coef, depth_exp, ptr, alive, omask)


# speedup vs baseline: 1062.4712x; 1.1974x over previous
import functools

import jax
import jax.numpy as jnp
from jax import lax
from jax.experimental import pallas as pl
from jax.experimental.pallas import tpu as pltpu
from jax.experimental.pallas import tpu_sc as plsc

N = 2048
T = 16
DT = 3600.0
L = 16
NW = 16
W = N // NW
LN2 = 0.6931471805599453
SQRT2 = 1.4142135623730951


def _vlog(x):
    bits = lax.bitcast_convert_type(x, jnp.int32)
    e = (bits >> 23) - 127
    m = lax.bitcast_convert_type((bits & 0x007FFFFF) | 0x3F800000,
                                 jnp.float32)
    big = m >= SQRT2
    m = jnp.where(big, m * 0.5, m)
    e = jnp.where(big, e + 1, e)
    s = (m - 1.0) * (1.0 / (m + 1.0))
    z = s * s
    p = 1.0 + z * (1.0/3.0 + z * (1.0/5.0 + z * (1.0/7.0 + z * (1.0/9.0))))
    return e.astype(jnp.float32) * LN2 + 2.0 * s * p


def _tsweep(body, *, carry=None, unroll=4):
    if carry is None:
        def wrapped(i, j):
            body(i)
            return j
        plsc.parallel_loop(0, W, step=L, unroll=unroll,
                           carry=jnp.int32(0))(wrapped)
        return None
    return plsc.parallel_loop(0, W, step=L, unroll=unroll, carry=carry)(body)


def _routing_body(lat_hbm, len_hbm, slope_hbm, n_hbm, wc_hbm, we_hbm, dc_hbm,
                  de_hbm, ptr_hbm, alive_hbm, omask_hbm, out_hbm,
                  shV, shU, shC0, shWa, shWb, shPa, shPb, shPart,
                  latloc, lenl, sql, invnl, ssll, wcl, wel, dcl, del_,
                  ptrl, alivel, omaskl,
                  Ql, Qpl, Ipl, C0l, vl, wl, pll, prod, gw, gp, flocal,
                  gflag, obuf, outacc):
    cid = lax.axis_index("c")
    sid = lax.axis_index("s")
    on = cid == 0
    lead = on & (sid == 0)
    base = sid * W
    sl_own = pl.ds(base, W)

    @pl.when(on)
    def _stage():
        pltpu.sync_copy(len_hbm.at[sl_own], lenl)
        pltpu.sync_copy(slope_hbm.at[sl_own], sql)
        pltpu.sync_copy(n_hbm.at[sl_own], invnl)
        pltpu.sync_copy(wc_hbm.at[sl_own], wcl)
        pltpu.sync_copy(we_hbm.at[sl_own], wel)
        pltpu.sync_copy(dc_hbm.at[sl_own], dcl)
        pltpu.sync_copy(de_hbm.at[sl_own], del_)
        pltpu.sync_copy(ptr_hbm.at[sl_own], ptrl)
        pltpu.sync_copy(alive_hbm.at[sl_own], alivel)
        pltpu.sync_copy(omask_hbm.at[sl_own], omaskl)

        def _pre(i):
            sl = pl.ds(i, L)
            slope_safe = jnp.maximum(sql[sl], 1e-6)
            sql[sl] = jnp.exp(0.5 * _vlog(slope_safe))
            invnl[sl] = 1.0 / jnp.maximum(invnl[sl], 0.001)
            ssll[sl] = slope_safe * lenl[sl]
            Ql[sl] = jnp.full((L,), 0.1, jnp.float32)
            Qpl[sl] = jnp.full((L,), 0.1, jnp.float32)
            Ipl[sl] = jnp.zeros((L,), jnp.float32)
        _tsweep(_pre, unroll=2)
        outacc[:] = jnp.zeros((L,), jnp.float32)

    def _timestep(t, _):
        @pl.when(on)
        def _a():
            pltpu.sync_copy(lat_hbm.at[t, sl_own], latloc)

            def _params(i):
                sl = pl.ds(i, L)
                Qr = jnp.maximum(Ql[sl], 0.1)
                logQ = _vlog(Qr)
                width = wcl[sl] * jnp.exp(wel[sl] * logQ)
                depth = dcl[sl] * jnp.exp(del_[sl] * logQ)
                Rh = (width * depth) * (1.0 / (width + 2.0 * depth))
                V = invnl[sl] * jnp.exp((2.0 / 3.0) * _vlog(Rh)) * sql[sl]
                cel = jnp.maximum((5.0 / 3.0) * V, 0.01)
                K = jnp.maximum(lenl[sl] * (1.0 / cel), DT * 0.1)
                X = 0.5 * (1.0 - Qr * (1.0 / (width * cel * ssll[sl] + 1e-6)))
                X = jnp.clip(X, 0.0, 0.5)
                kx2 = 2.0 * K * X
                d2 = 2.0 * K * (1.0 - X)
                rden = 1.0 / (d2 + DT)
                c0 = jnp.maximum((DT - kx2) * rden, 0.0)
                c1 = jnp.maximum((DT + kx2) * rden, 0.0)
                c2 = jnp.maximum((d2 - DT) * rden, 0.0)
                rtot = 1.0 / (c0 + c1 + c2)
                c0 = c0 * rtot
                C0l[sl] = c0
                vl[sl] = (c0 * latloc[sl] + (c1 * rtot) * Ipl[sl]
                          + (c2 * rtot) * Qpl[sl])
            _tsweep(_params)
            pltpu.sync_copy(C0l, shC0.at[sl_own])

        plsc.subcore_barrier()

        @pl.when(on)
        def _b():
            pltpu.sync_copy(shC0.at[ptrl], gw)

            def _winit(i):
                sl = pl.ds(i, L)
                wl[sl] = gw[sl] * alivel[sl]
                pll[sl] = ptrl[sl]
            _tsweep(_winit)
            pltpu.sync_copy(wl, shWa.at[sl_own])
            pltpu.sync_copy(ptrl, shPa.at[sl_own])

        plsc.subcore_barrier()

        for k in range(11):
            shWs, shPs = (shWa, shPa) if k % 2 == 0 else (shWb, shPb)
            shWd, shPd = (shWb, shPb) if k % 2 == 0 else (shWa, shPa)
            if k < 6:
                g = on
            else:
                g = on & (jnp.sum(gflag[:], axis=0) > 0.0)

            @pl.when(g)
            def _init():
                pltpu.sync_copy(vl, shV.at[sl_own])

            plsc.subcore_barrier()

            @pl.when(g)
            def _scsq(k=k, shWs=shWs, shPs=shPs, shWd=shWd, shPd=shPd):
                def _prod(i):
                    sl = pl.ds(i, L)
                    prod[sl] = wl[sl] * vl[sl]
                _tsweep(_prod)
                pltpu.sync_copy(prod, shV.at[pll], add=True)
                if k < 10:
                    pltpu.sync_copy(shWs.at[pll], gw)
                    pltpu.sync_copy(shPs.at[pll], gp)

                    def _sq(i, acc):
                        sl = pl.ds(i, L)
                        w2 = wl[sl] * gw[sl]
                        wl[sl] = w2
                        pll[sl] = gp[sl]
                        return acc + w2
                    acc = _tsweep(_sq, carry=jnp.zeros((L,), jnp.float32))
                    pltpu.sync_copy(wl, shWd.at[sl_own])
                    pltpu.sync_copy(pll, shPd.at[sl_own])
                    if k >= 5:
                        obuf[:] = acc
                        pltpu.sync_copy(obuf, shPart.at[pl.ds(sid * L, L)])

            plsc.subcore_barrier()

            @pl.when(g)
            def _pull(k=k):
                pltpu.sync_copy(shV.at[sl_own], vl)
                if 5 <= k < 10:
                    pltpu.sync_copy(shPart, flocal)

                    def _fsum(i, acc):
                        return acc + flocal[pl.ds(i, L)]
                    facc = plsc.parallel_loop(
                        0, NW * L, step=L, unroll=4,
                        carry=jnp.zeros((L,), jnp.float32))(_fsum)
                    gflag[:] = facc

            plsc.subcore_barrier()

        @pl.when(on)
        def _c():
            pltpu.sync_copy(latloc, shU.at[sl_own])

        plsc.subcore_barrier()

        @pl.when(on)
        def _d():
            def _prod2(i):
                sl = pl.ds(i, L)
                prod[sl] = alivel[sl] * vl[sl]
            _tsweep(_prod2)
            pltpu.sync_copy(prod, shU.at[ptrl], add=True)

        plsc.subcore_barrier()

        @pl.when(on)
        def _e():
            pltpu.sync_copy(shU.at[sl_own], Ipl)

            def _upd(i, acc):
                sl = pl.ds(i, L)
                vc = vl[sl]
                acc = acc + omaskl[sl] * vc
                Qpl[sl] = Ql[sl]
                Ql[sl] = vc
                return acc
            acc = _tsweep(_upd, carry=jnp.zeros((L,), jnp.float32))
            obuf[:] = acc
            pltpu.sync_copy(obuf, shPart.at[pl.ds(sid * L, L)])

        plsc.subcore_barrier()

        @pl.when(on)
        def _f(t=t):
            pltpu.sync_copy(shPart, flocal)

            def _osum(i, acc):
                return acc + flocal[pl.ds(i, L)]
            acc = plsc.parallel_loop(
                0, NW * L, step=L, unroll=4,
                carry=jnp.zeros((L,), jnp.float32))(_osum)
            s = jnp.sum(acc, axis=0)
            tlane = lax.iota(jnp.int32, L) == t
            outacc[:] = outacc[:] + jnp.where(tlane, s, 0.0)

        plsc.subcore_barrier()
        return 0

    lax.fori_loop(0, T, _timestep, 0, unroll=False)

    @pl.when(lead)
    def _out():
        pltpu.sync_copy(outacc, out_hbm)


def kernel(lateral_inflows, lengths, slopes, manning_n, width_coef, width_exp,
           depth_coef, depth_exp, upstream_mask, is_outlet):
    ptr = jnp.argmax(upstream_mask, axis=0).astype(jnp.int32)
    alive = jnp.any(upstream_mask, axis=0).astype(jnp.float32)
    omask = is_outlet.astype(jnp.float32)

    shf = pltpu.VMEM_SHARED((N,), jnp.float32)
    shi = pltpu.VMEM_SHARED((N,), jnp.int32)
    locf = pltpu.VMEM((W,), jnp.float32)
    loci = pltpu.VMEM((W,), jnp.int32)
    run = pl.kernel(
        _routing_body,
        out_type=jax.ShapeDtypeStruct((T,), jnp.float32),
        mesh=plsc.VectorSubcoreMesh(core_axis_name="c", subcore_axis_name="s"),
        compiler_params=pltpu.CompilerParams(needs_layout_passes=False),
        scratch_types=[
            shf, shf, shf,
            shf, shf, shi, shi,
            pltpu.VMEM_SHARED((NW * L,), jnp.float32),
            locf, locf, locf, locf, locf,
            locf, locf, locf, locf,
            loci, locf, locf,
            locf, locf, locf, locf,
            locf, locf, loci, locf,
            locf, loci,
            pltpu.VMEM((NW * L,), jnp.float32),
            pltpu.VMEM((L,), jnp.float32),
            pltpu.VMEM((L,), jnp.float32),
            pltpu.VMEM((L,), jnp.float32),
        ],
    )
    return run(lateral_inflows.astype(jnp.float32), lengths, slopes, manning_n,
               width_coef, width_exp, depth_coef, depth_exp, ptr, alive, omask)


# speedup vs baseline: 1174.6642x; 1.1056x over previous
import functools

import jax
import jax.numpy as jnp
from jax import lax
from jax.experimental import pallas as pl
from jax.experimental.pallas import tpu as pltpu
from jax.experimental.pallas import tpu_sc as plsc

N = 2048
T = 16
DT = 3600.0
L = 16
NW = 16
W = N // NW
LN2 = 0.6931471805599453
SQRT2 = 1.4142135623730951


def _vlog(x):
    bits = lax.bitcast_convert_type(x, jnp.int32)
    e = (bits >> 23) - 127
    m = lax.bitcast_convert_type((bits & 0x007FFFFF) | 0x3F800000,
                                 jnp.float32)
    big = m >= SQRT2
    m = jnp.where(big, m * 0.5, m)
    e = jnp.where(big, e + 1, e)
    s = (m - 1.0) * (1.0 / (m + 1.0))
    z = s * s
    p = 1.0 + z * (1.0/3.0 + z * (1.0/5.0 + z * (1.0/7.0 + z * (1.0/9.0))))
    return e.astype(jnp.float32) * LN2 + 2.0 * s * p


def _tsweep(body, *, carry=None, unroll=4):
    if carry is None:
        def wrapped(i, j):
            body(i)
            return j
        plsc.parallel_loop(0, W, step=L, unroll=unroll,
                           carry=jnp.int32(0))(wrapped)
        return None
    return plsc.parallel_loop(0, W, step=L, unroll=unroll, carry=carry)(body)


def _partsum(flocal):
    def _fsum(i, acc):
        return acc + flocal[pl.ds(i, L)]
    return plsc.parallel_loop(0, NW * L, step=L, unroll=4,
                              carry=jnp.zeros((L,), jnp.float32))(_fsum)


def _routing_body(lat_hbm, len_hbm, slope_hbm, n_hbm, wc_hbm, we_hbm, dc_hbm,
                  de_hbm, ptr_hbm, alive_hbm, omask_hbm, out_hbm,
                  shV, shU, shC0, shWa, shWb, shPa, shPb, shPart,
                  latloc, lenl, sql, invnl, ssll, wcl, wel, dcl, del_,
                  ptrl, alivel, omaskl,
                  Ql, Qpl, Ipl, C0l, vl, wl, pll, prod, gw, gp, flocal,
                  gflag, obuf, outacc):
    cid = lax.axis_index("c")
    sid = lax.axis_index("s")
    lead = (cid == 0) & (sid == 0)
    base = sid * W
    sl_own = pl.ds(base, W)

    pltpu.sync_copy(len_hbm.at[sl_own], lenl)
    pltpu.sync_copy(slope_hbm.at[sl_own], sql)
    pltpu.sync_copy(n_hbm.at[sl_own], invnl)
    pltpu.sync_copy(wc_hbm.at[sl_own], wcl)
    pltpu.sync_copy(we_hbm.at[sl_own], wel)
    pltpu.sync_copy(dc_hbm.at[sl_own], dcl)
    pltpu.sync_copy(de_hbm.at[sl_own], del_)
    pltpu.sync_copy(ptr_hbm.at[sl_own], ptrl)
    pltpu.sync_copy(alive_hbm.at[sl_own], alivel)
    pltpu.sync_copy(omask_hbm.at[sl_own], omaskl)

    def _pre(i):
        sl = pl.ds(i, L)
        slope_safe = jnp.maximum(sql[sl], 1e-6)
        sql[sl] = jnp.exp(0.5 * _vlog(slope_safe))
        invnl[sl] = 1.0 / jnp.maximum(invnl[sl], 0.001)
        ssll[sl] = slope_safe * lenl[sl]
        Ql[sl] = jnp.full((L,), 0.1, jnp.float32)
        Qpl[sl] = jnp.full((L,), 0.1, jnp.float32)
        Ipl[sl] = jnp.zeros((L,), jnp.float32)
    _tsweep(_pre, unroll=2)
    outacc[:] = jnp.zeros((L,), jnp.float32)
    gflag[:] = jnp.full((L,), 1.0, jnp.float32)

    def _timestep(t, _):
        pltpu.sync_copy(lat_hbm.at[t, sl_own], latloc)

        def _params(i):
            sl = pl.ds(i, L)
            Qr = jnp.maximum(Ql[sl], 0.1)
            logQ = _vlog(Qr)
            width = wcl[sl] * jnp.exp(wel[sl] * logQ)
            depth = dcl[sl] * jnp.exp(del_[sl] * logQ)
            Rh = (width * depth) * (1.0 / (width + 2.0 * depth))
            V = invnl[sl] * jnp.exp((2.0 / 3.0) * _vlog(Rh)) * sql[sl]
            cel = jnp.maximum((5.0 / 3.0) * V, 0.01)
            K = jnp.maximum(lenl[sl] * (1.0 / cel), DT * 0.1)
            X = 0.5 * (1.0 - Qr * (1.0 / (width * cel * ssll[sl] + 1e-6)))
            X = jnp.clip(X, 0.0, 0.5)
            kx2 = 2.0 * K * X
            d2 = 2.0 * K * (1.0 - X)
            rden = 1.0 / (d2 + DT)
            c0 = jnp.maximum((DT - kx2) * rden, 0.0)
            c1 = jnp.maximum((DT + kx2) * rden, 0.0)
            c2 = jnp.maximum((d2 - DT) * rden, 0.0)
            rtot = 1.0 / (c0 + c1 + c2)
            c0 = c0 * rtot
            C0l[sl] = c0
            vl[sl] = (c0 * latloc[sl] + (c1 * rtot) * Ipl[sl]
                      + (c2 * rtot) * Qpl[sl])
        _tsweep(_params)
        pltpu.sync_copy(C0l, shC0.at[sl_own])

        plsc.subcore_barrier()

        pltpu.sync_copy(shC0.at[ptrl], gw)

        def _winit(i):
            sl = pl.ds(i, L)
            wl[sl] = gw[sl] * alivel[sl]
            pll[sl] = ptrl[sl]
        _tsweep(_winit)
        pltpu.sync_copy(wl, shWa.at[sl_own])
        pltpu.sync_copy(ptrl, shPa.at[sl_own])
        pltpu.sync_copy(vl, shV.at[sl_own])
        pltpu.sync_copy(latloc, shU.at[sl_own])

        plsc.subcore_barrier()

        for k in range(11):
            shWs, shPs = (shWa, shPa) if k % 2 == 0 else (shWb, shPb)
            shWd, shPd = (shWb, shPb) if k % 2 == 0 else (shWa, shPa)
            def _round(k=k, shWs=shWs, shPs=shPs, shWd=shWd, shPd=shPd):
                def _prod(i):
                    sl = pl.ds(i, L)
                    prod[sl] = wl[sl] * vl[sl]
                _tsweep(_prod)
                pltpu.sync_copy(prod, shV.at[pll], add=True)
                if k < 10:
                    pltpu.sync_copy(shWs.at[pll], gw)
                    pltpu.sync_copy(shPs.at[pll], gp)

                    def _sq(i, acc):
                        sl = pl.ds(i, L)
                        w2 = wl[sl] * gw[sl]
                        wl[sl] = w2
                        pll[sl] = gp[sl]
                        return acc + w2
                    acc = _tsweep(_sq, carry=jnp.zeros((L,), jnp.float32))
                    pltpu.sync_copy(wl, shWd.at[sl_own])
                    pltpu.sync_copy(pll, shPd.at[sl_own])
                    if k >= 5:
                        obuf[:] = acc
                        pltpu.sync_copy(obuf, shPart.at[pl.ds(sid * L, L)])

                plsc.subcore_barrier()

                pltpu.sync_copy(shV.at[sl_own], vl)
                if 5 <= k < 10:
                    pltpu.sync_copy(shPart, flocal)
                    gflag[:] = _partsum(flocal)

                plsc.subcore_barrier()

            if k < 6:
                _round()
            else:
                pl.when(jnp.sum(gflag[:], axis=0) > 0.0)(_round)

        def _prod2(i):
            sl = pl.ds(i, L)
            prod[sl] = alivel[sl] * vl[sl]
        _tsweep(_prod2)
        pltpu.sync_copy(prod, shU.at[ptrl], add=True)

        plsc.subcore_barrier()

        pltpu.sync_copy(shU.at[sl_own], Ipl)

        def _upd(i, acc):
            sl = pl.ds(i, L)
            vc = vl[sl]
            acc = acc + omaskl[sl] * vc
            Qpl[sl] = Ql[sl]
            Ql[sl] = vc
            return acc
        acc = _tsweep(_upd, carry=jnp.zeros((L,), jnp.float32))
        obuf[:] = acc
        pltpu.sync_copy(obuf, shPart.at[pl.ds(sid * L, L)])

        plsc.subcore_barrier()

        pltpu.sync_copy(shPart, flocal)
        s = jnp.sum(_partsum(flocal), axis=0)
        tlane = lax.iota(jnp.int32, L) == t
        outacc[:] = outacc[:] + jnp.where(tlane, s, 0.0)
        return 0

    lax.fori_loop(0, T, _timestep, 0, unroll=False)

    @pl.when(lead)
    def _out():
        pltpu.sync_copy(outacc, out_hbm)


def kernel(lateral_inflows, lengths, slopes, manning_n, width_coef, width_exp,
           depth_coef, depth_exp, upstream_mask, is_outlet):
    ptr = jnp.argmax(upstream_mask, axis=0).astype(jnp.int32)
    alive = jnp.any(upstream_mask, axis=0).astype(jnp.float32)
    omask = is_outlet.astype(jnp.float32)

    shf = pltpu.VMEM_SHARED((N,), jnp.float32)
    shi = pltpu.VMEM_SHARED((N,), jnp.int32)
    locf = pltpu.VMEM((W,), jnp.float32)
    loci = pltpu.VMEM((W,), jnp.int32)
    run = pl.kernel(
        _routing_body,
        out_type=jax.ShapeDtypeStruct((T,), jnp.float32),
        mesh=plsc.VectorSubcoreMesh(core_axis_name="c", subcore_axis_name="s"),
        compiler_params=pltpu.CompilerParams(needs_layout_passes=False),
        scratch_types=[
            shf, shf, shf,
            shf, shf, shi, shi,
            pltpu.VMEM_SHARED((NW * L,), jnp.float32),
            locf, locf, locf, locf, locf,
            locf, locf, locf, locf,
            loci, locf, locf,
            locf, locf, locf, locf,
            locf, locf, loci, locf,
            locf, loci,
            pltpu.VMEM((NW * L,), jnp.float32),
            pltpu.VMEM((L,), jnp.float32),
            pltpu.VMEM((L,), jnp.float32),
            pltpu.VMEM((L,), jnp.float32),
        ],
    )
    return run(lateral_inflows.astype(jnp.float32), lengths, slopes, manning_n,
               width_coef, width_exp, depth_coef, depth_exp, ptr, alive, omask)


# speedup vs baseline: 1325.2301x; 1.1282x over previous
import functools

import jax
import jax.numpy as jnp
from jax import lax
from jax.experimental import pallas as pl
from jax.experimental.pallas import tpu as pltpu
from jax.experimental.pallas import tpu_sc as plsc

N = 2048
T = 16
DT = 3600.0
L = 16
NW = 16
W = N // NW
LN2 = 0.6931471805599453
SQRT2 = 1.4142135623730951


def _vlog(x):
    bits = lax.bitcast_convert_type(x, jnp.int32)
    e = (bits >> 23) - 127
    m = lax.bitcast_convert_type((bits & 0x007FFFFF) | 0x3F800000,
                                 jnp.float32)
    big = m >= SQRT2
    m = jnp.where(big, m * 0.5, m)
    e = jnp.where(big, e + 1, e)
    s = (m - 1.0) * (1.0 / (m + 1.0))
    z = s * s
    p = 1.0 + z * (1.0/3.0 + z * (1.0/5.0 + z * (1.0/7.0 + z * (1.0/9.0))))
    return e.astype(jnp.float32) * LN2 + 2.0 * s * p


def _tsweep(body, *, carry=None, unroll=4):
    if carry is None:
        def wrapped(i, j):
            body(i)
            return j
        plsc.parallel_loop(0, W, step=L, unroll=unroll,
                           carry=jnp.int32(0))(wrapped)
        return None
    return plsc.parallel_loop(0, W, step=L, unroll=unroll, carry=carry)(body)


def _partsum(flocal):
    def _fsum(i, acc):
        return acc + flocal[pl.ds(i, L)]
    return plsc.parallel_loop(0, NW * L, step=L, unroll=4,
                              carry=jnp.zeros((L,), jnp.float32))(_fsum)


def _routing_body(lat_hbm, len_hbm, slope_hbm, n_hbm, wc_hbm, we_hbm, dc_hbm,
                  de_hbm, ptr_hbm, alive_hbm, omask_hbm, out_hbm,
                  shV, shU, shC0, shWa, shWb, shPa, shPb, shPart,
                  latloc, lenl, sql, invnl, ssll, wcl, wel, dcl, del_,
                  ptrl, alivel, omaskl,
                  Ql, Qpl, Ipl, C0l, vl, wl, pll, prod, gw, gp, flocal,
                  gflag, obuf, outacc, dsem):
    cid = lax.axis_index("c")
    sid = lax.axis_index("s")
    lead = (cid == 0) & (sid == 0)
    base = sid * W
    sl_own = pl.ds(base, W)

    pltpu.sync_copy(len_hbm.at[sl_own], lenl)
    pltpu.sync_copy(slope_hbm.at[sl_own], sql)
    pltpu.sync_copy(n_hbm.at[sl_own], invnl)
    pltpu.sync_copy(wc_hbm.at[sl_own], wcl)
    pltpu.sync_copy(we_hbm.at[sl_own], wel)
    pltpu.sync_copy(dc_hbm.at[sl_own], dcl)
    pltpu.sync_copy(de_hbm.at[sl_own], del_)
    pltpu.sync_copy(ptr_hbm.at[sl_own], ptrl)
    pltpu.sync_copy(alive_hbm.at[sl_own], alivel)
    pltpu.sync_copy(omask_hbm.at[sl_own], omaskl)

    def _pre(i):
        sl = pl.ds(i, L)
        slope_safe = jnp.maximum(sql[sl], 1e-6)
        sql[sl] = jnp.exp(0.5 * _vlog(slope_safe))
        invnl[sl] = 1.0 / jnp.maximum(invnl[sl], 0.001)
        ssll[sl] = slope_safe * lenl[sl]
        Ql[sl] = jnp.full((L,), 0.1, jnp.float32)
        Qpl[sl] = jnp.full((L,), 0.1, jnp.float32)
        Ipl[sl] = jnp.zeros((L,), jnp.float32)
    _tsweep(_pre, unroll=2)
    outacc[:] = jnp.zeros((L,), jnp.float32)
    gflag[:] = jnp.full((L,), 1.0, jnp.float32)

    def _timestep(t, _):
        pltpu.sync_copy(lat_hbm.at[t, sl_own], latloc)

        def _params(i):
            sl = pl.ds(i, L)
            Qr = jnp.maximum(Ql[sl], 0.1)
            logQ = _vlog(Qr)
            width = wcl[sl] * jnp.exp(wel[sl] * logQ)
            depth = dcl[sl] * jnp.exp(del_[sl] * logQ)
            Rh = (width * depth) * (1.0 / (width + 2.0 * depth))
            V = invnl[sl] * jnp.exp((2.0 / 3.0) * _vlog(Rh)) * sql[sl]
            cel = jnp.maximum((5.0 / 3.0) * V, 0.01)
            K = jnp.maximum(lenl[sl] * (1.0 / cel), DT * 0.1)
            X = 0.5 * (1.0 - Qr * (1.0 / (width * cel * ssll[sl] + 1e-6)))
            X = jnp.clip(X, 0.0, 0.5)
            kx2 = 2.0 * K * X
            d2 = 2.0 * K * (1.0 - X)
            rden = 1.0 / (d2 + DT)
            c0 = jnp.maximum((DT - kx2) * rden, 0.0)
            c1 = jnp.maximum((DT + kx2) * rden, 0.0)
            c2 = jnp.maximum((d2 - DT) * rden, 0.0)
            rtot = 1.0 / (c0 + c1 + c2)
            c0 = c0 * rtot
            C0l[sl] = c0
            vl[sl] = (c0 * latloc[sl] + (c1 * rtot) * Ipl[sl]
                      + (c2 * rtot) * Qpl[sl])
        _tsweep(_params)
        pltpu.sync_copy(C0l, shC0.at[sl_own])

        plsc.subcore_barrier()

        pltpu.sync_copy(shC0.at[ptrl], gw)

        def _winit(i):
            sl = pl.ds(i, L)
            wl[sl] = gw[sl] * alivel[sl]
            pll[sl] = ptrl[sl]
        _tsweep(_winit)
        pltpu.sync_copy(wl, shWa.at[sl_own])
        pltpu.sync_copy(ptrl, shPa.at[sl_own])
        pltpu.sync_copy(vl, shV.at[sl_own])
        pltpu.sync_copy(latloc, shU.at[sl_own])

        plsc.subcore_barrier()

        for k in range(11):
            shWs, shPs = (shWa, shPa) if k % 2 == 0 else (shWb, shPb)
            shWd, shPd = (shWb, shPb) if k % 2 == 0 else (shWa, shPa)
            def _round(k=k, shWs=shWs, shPs=shPs, shWd=shWd, shPd=shPd):
                def _prod(i):
                    sl = pl.ds(i, L)
                    prod[sl] = wl[sl] * vl[sl]
                _tsweep(_prod)
                d1 = pltpu.make_async_copy(prod, shV.at[pll], dsem)
                d1.start(add=True)
                if k < 10:
                    d2 = pltpu.make_async_copy(shWs.at[pll], gw, dsem)
                    d3 = pltpu.make_async_copy(shPs.at[pll], gp, dsem)
                    d2.start()
                    d3.start()
                    d1.wait()
                    d2.wait()
                    d3.wait()

                    def _sq(i, acc):
                        sl = pl.ds(i, L)
                        w2 = wl[sl] * gw[sl]
                        wl[sl] = w2
                        pll[sl] = gp[sl]
                        return acc + w2
                    acc = _tsweep(_sq, carry=jnp.zeros((L,), jnp.float32))
                    pltpu.sync_copy(wl, shWd.at[sl_own])
                    pltpu.sync_copy(pll, shPd.at[sl_own])
                    if k >= 5:
                        obuf[:] = acc
                        pltpu.sync_copy(obuf, shPart.at[pl.ds(sid * L, L)])
                else:
                    d1.wait()

                plsc.subcore_barrier()

                pltpu.sync_copy(shV.at[sl_own], vl)
                if 5 <= k < 10:
                    pltpu.sync_copy(shPart, flocal)
                    gflag[:] = _partsum(flocal)

                plsc.subcore_barrier()

            if k < 6:
                _round()
            else:
                pl.when(jnp.sum(gflag[:], axis=0) > 0.0)(_round)

        def _prod2(i):
            sl = pl.ds(i, L)
            prod[sl] = alivel[sl] * vl[sl]
        _tsweep(_prod2)
        du = pltpu.make_async_copy(prod, shU.at[ptrl], dsem)
        du.start(add=True)

        def _upd(i, acc):
            sl = pl.ds(i, L)
            vc = vl[sl]
            acc = acc + omaskl[sl] * vc
            Qpl[sl] = Ql[sl]
            Ql[sl] = vc
            return acc
        acc = _tsweep(_upd, carry=jnp.zeros((L,), jnp.float32))
        obuf[:] = acc
        pltpu.sync_copy(obuf, shPart.at[pl.ds(sid * L, L)])
        du.wait()

        plsc.subcore_barrier()

        pltpu.sync_copy(shU.at[sl_own], Ipl)
        pltpu.sync_copy(shPart, flocal)
        s = jnp.sum(_partsum(flocal), axis=0)
        tlane = lax.iota(jnp.int32, L) == t
        outacc[:] = outacc[:] + jnp.where(tlane, s, 0.0)
        return 0

    lax.fori_loop(0, T, _timestep, 0, unroll=False)

    @pl.when(lead)
    def _out():
        pltpu.sync_copy(outacc, out_hbm)


def kernel(lateral_inflows, lengths, slopes, manning_n, width_coef, width_exp,
           depth_coef, depth_exp, upstream_mask, is_outlet):
    ptr = jnp.argmax(upstream_mask, axis=0).astype(jnp.int32)
    alive = jnp.any(upstream_mask, axis=0).astype(jnp.float32)
    omask = is_outlet.astype(jnp.float32)

    shf = pltpu.VMEM_SHARED((N,), jnp.float32)
    shi = pltpu.VMEM_SHARED((N,), jnp.int32)
    locf = pltpu.VMEM((W,), jnp.float32)
    loci = pltpu.VMEM((W,), jnp.int32)
    run = pl.kernel(
        _routing_body,
        out_type=jax.ShapeDtypeStruct((T,), jnp.float32),
        mesh=plsc.VectorSubcoreMesh(core_axis_name="c", subcore_axis_name="s"),
        compiler_params=pltpu.CompilerParams(needs_layout_passes=False),
        scratch_types=[
            shf, shf, shf,
            shf, shf, shi, shi,
            pltpu.VMEM_SHARED((NW * L,), jnp.float32),
            locf, locf, locf, locf, locf,
            locf, locf, locf, locf,
            loci, locf, locf,
            locf, locf, locf, locf,
            locf, locf, loci, locf,
            locf, loci,
            pltpu.VMEM((NW * L,), jnp.float32),
            pltpu.VMEM((L,), jnp.float32),
            pltpu.VMEM((L,), jnp.float32),
            pltpu.VMEM((L,), jnp.float32),
            pltpu.SemaphoreType.DMA,
        ],
    )
    return run(lateral_inflows.astype(jnp.float32), lengths, slopes, manning_n,
               width_coef, width_exp, depth_coef, depth_exp, ptr, alive, omask)
